# baseline (speedup 1.0000x reference)
import functools

import jax
import jax.numpy as jnp
from jax import lax
from jax.experimental import pallas as pl
from jax.experimental.pallas import tpu as pltpu
from jax.experimental.pallas import tpu_sc as plsc

_EMBED = 32
_NC = 2
_NS = 16
_NW = _NC * _NS
_CHUNK = 800
_NBUF = 4


@functools.lru_cache(maxsize=None)
def _build(B: int):
    b_per_w = B // _NW
    n_chunks = b_per_w // _CHUNK
    assert B % (8 * _NW) == 0 and b_per_w % _CHUNK == 0
    assert n_chunks % _NBUF == 0 and n_chunks // _NBUF >= 2

    mesh = plsc.VectorSubcoreMesh(core_axis_name="c", subcore_axis_name="s")

    scratch = (
        [pltpu.VMEM((_CHUNK,), jnp.int32) for _ in range(_NBUF)]
        + [pltpu.VMEM((_CHUNK, _EMBED), jnp.float32) for _ in range(_NBUF)]
        + [pltpu.SemaphoreType.DMA for _ in range(2 * _NBUF)]
    )

    @functools.partial(
        pl.kernel,
        mesh=mesh,
        out_type=jax.ShapeDtypeStruct((B, _EMBED), jnp.float32),
        compiler_params=pltpu.CompilerParams(use_tc_tiling_on_sc=False),
        scratch_types=scratch,
    )
    def gather_k(idx_hbm, table_hbm, out_hbm, *refs):
        idx_v = refs[0:_NBUF]
        rows_v = refs[_NBUF:2 * _NBUF]
        gsem = refs[2 * _NBUF:3 * _NBUF]
        ssem = refs[3 * _NBUF:4 * _NBUF]

        wid = lax.axis_index("s") * _NC + lax.axis_index("c")
        base = wid * b_per_w

        def out_slice(g):
            off = base + g * _CHUNK
            return out_hbm.at[pl.ds(off, _CHUNK)]

        def fire(g, s, wait_store):
            if wait_store:
                pltpu.make_async_copy(
                    rows_v[s], out_slice(g - _NBUF), ssem[s]).wait()
            off = base + g * _CHUNK
            pltpu.sync_copy(idx_hbm.at[pl.ds(off, _CHUNK)], idx_v[s])
            pltpu.async_copy(table_hbm.at[idx_v[s]], rows_v[s], gsem[s])

        def drain(g, s):
            pltpu.make_async_copy(
                table_hbm.at[idx_v[s]], rows_v[s], gsem[s]).wait()
            pltpu.async_copy(rows_v[s], out_slice(g), ssem[s])

        for s in range(_NBUF):
            fire(s, s, wait_store=False)

        def body(j, carry):
            g0 = j * _NBUF
            for s in range(_NBUF):
                drain(g0 + s, s)
            for s in range(_NBUF):
                fire(g0 + _NBUF + s, s, wait_store=True)
            return carry

        lax.fori_loop(0, n_chunks // _NBUF - 1, body, 0)

        g0 = n_chunks - _NBUF
        for s in range(_NBUF):
            drain(g0 + s, s)
        for s in range(_NBUF):
            pltpu.make_async_copy(
                rows_v[s], out_slice(g0 + s), ssem[s]).wait()

    return gather_k


_BB = 16


@functools.lru_cache(maxsize=None)
def _tc_convert(batch: int, hist: int):
    w = hist * _EMBED // 128
    f = 128 // _EMBED

    def conv_kernel(x_ref, o_ref):
        x = x_ref[...]
        o_ref[...] = jnp.concatenate(
            [x[:, :, j * _EMBED:(j + 1) * _EMBED] for j in range(f)], axis=1)

    return pl.pallas_call(
        conv_kernel,
        grid=(batch // _BB,),
        in_specs=[pl.BlockSpec((_BB, w, 128), lambda i: (i, 0, 0))],
        out_specs=pl.BlockSpec((_BB, hist, _EMBED), lambda i: (i, 0, 0)),
        out_shape=jax.ShapeDtypeStruct((batch, hist, _EMBED), jnp.float32),
    )


def kernel(a, table):
    batch, hist = a.shape
    B = batch * hist
    w = hist * _EMBED // 128
    f = 128 // _EMBED
    ap = a.reshape(batch, f, w).transpose(0, 2, 1).reshape(batch, hist)
    idx = ap.reshape(B).astype(jnp.int32)
    out = _build(B)(idx, table)
    x = out.reshape(batch, w, 128)
    return _tc_convert(batch, hist)(x)


# speedup vs baseline: 2.8714x; 2.8714x over previous
import functools

import jax
import jax.numpy as jnp
from jax import lax
from jax.experimental import pallas as pl
from jax.experimental.pallas import tpu as pltpu
from jax.experimental.pallas import tpu_sc as plsc

_EMBED = 32
_NC = 2
_NS = 16
_NW = _NC * _NS
_CHUNK = 800
_NBUF = 4


@functools.lru_cache(maxsize=None)
def _build(B: int):
    b_per_w = B // _NW
    n_chunks = b_per_w // _CHUNK
    assert B % (8 * _NW) == 0 and b_per_w % _CHUNK == 0
    assert n_chunks % _NBUF == 0 and n_chunks // _NBUF >= 2

    mesh = plsc.VectorSubcoreMesh(core_axis_name="c", subcore_axis_name="s")

    scratch = (
        [pltpu.VMEM((_CHUNK,), jnp.int32) for _ in range(_NBUF)]
        + [pltpu.VMEM((_CHUNK, _EMBED), jnp.float32) for _ in range(_NBUF)]
        + [pltpu.SemaphoreType.DMA for _ in range(2 * _NBUF)]
    )

    @functools.partial(
        pl.kernel,
        mesh=mesh,
        out_type=jax.ShapeDtypeStruct((B, _EMBED), jnp.float32),
        compiler_params=pltpu.CompilerParams(use_tc_tiling_on_sc=False),
        scratch_types=scratch,
    )
    def gather_k(idx_hbm, table_hbm, out_hbm, *refs):
        idx_v = refs[0:_NBUF]
        rows_v = refs[_NBUF:2 * _NBUF]
        gsem = refs[2 * _NBUF:3 * _NBUF]
        ssem = refs[3 * _NBUF:4 * _NBUF]

        wid = lax.axis_index("s") * _NC + lax.axis_index("c")
        base = wid * b_per_w

        def out_slice(g):
            off = base + g * _CHUNK
            return out_hbm.at[pl.ds(off, _CHUNK)]

        def fire(g, s, wait_store):
            if wait_store:
                pltpu.make_async_copy(
                    rows_v[s], out_slice(g - _NBUF), ssem[s]).wait()
            off = base + g * _CHUNK
            pltpu.sync_copy(idx_hbm.at[pl.ds(off, _CHUNK)], idx_v[s])
            pltpu.async_copy(table_hbm.at[idx_v[s]], rows_v[s], gsem[s])

        def drain(g, s):
            pltpu.make_async_copy(
                table_hbm.at[idx_v[s]], rows_v[s], gsem[s]).wait()
            pltpu.async_copy(rows_v[s], out_slice(g), ssem[s])

        for s in range(_NBUF):
            fire(s, s, wait_store=False)

        def body(j, carry):
            g0 = j * _NBUF
            for s in range(_NBUF):
                drain(g0 + s, s)
            for s in range(_NBUF):
                fire(g0 + _NBUF + s, s, wait_store=True)
            return carry

        lax.fori_loop(0, n_chunks // _NBUF - 1, body, 0)

        g0 = n_chunks - _NBUF
        for s in range(_NBUF):
            drain(g0 + s, s)
        for s in range(_NBUF):
            pltpu.make_async_copy(
                rows_v[s], out_slice(g0 + s), ssem[s]).wait()

    return gather_k


@functools.lru_cache(maxsize=None)
def _tc_transpose(rows: int, cols: int, bc: int):
    def tkern(x_ref, o_ref):
        o_ref[...] = x_ref[...].T

    return pl.pallas_call(
        tkern,
        grid=(rows // bc,),
        in_specs=[pl.BlockSpec((bc, cols), lambda i: (i, 0))],
        out_specs=pl.BlockSpec((cols, bc), lambda i: (0, i)),
        out_shape=jax.ShapeDtypeStruct((cols, rows), jnp.float32),
    )


def kernel(a, table):
    batch, hist = a.shape
    B = batch * hist
    idx = a.reshape(B).astype(jnp.int32)
    out = _build(B)(idx, table)
    x = out.reshape(batch, hist * _EMBED)
    t = _tc_transpose(batch, hist * _EMBED, 128)(x)
    return jnp.transpose(t.reshape(hist, _EMBED, batch), (2, 0, 1))


# speedup vs baseline: 2.9562x; 1.0296x over previous
import functools

import jax
import jax.numpy as jnp
from jax import lax
from jax.experimental import pallas as pl
from jax.experimental.pallas import tpu as pltpu
from jax.experimental.pallas import tpu_sc as plsc

_EMBED = 32
_NC = 2
_NS = 16
_NW = _NC * _NS
_CHUNK = 800
_NBUF = 4


@functools.lru_cache(maxsize=None)
def _build(B: int):
    b_per_w = B // _NW
    n_chunks = b_per_w // _CHUNK
    assert B % (8 * _NW) == 0 and b_per_w % _CHUNK == 0
    assert n_chunks % _NBUF == 0 and n_chunks // _NBUF >= 2

    mesh = plsc.VectorSubcoreMesh(core_axis_name="c", subcore_axis_name="s")

    scratch = (
        [pltpu.VMEM((_CHUNK,), jnp.int32) for _ in range(_NBUF)]
        + [pltpu.VMEM((_CHUNK, _EMBED), jnp.float32) for _ in range(_NBUF)]
        + [pltpu.SemaphoreType.DMA for _ in range(2 * _NBUF)]
    )

    @functools.partial(
        pl.kernel,
        mesh=mesh,
        out_type=jax.ShapeDtypeStruct((B, _EMBED), jnp.float32),
        compiler_params=pltpu.CompilerParams(use_tc_tiling_on_sc=False),
        scratch_types=scratch,
    )
    def gather_k(idx_hbm, table_hbm, out_hbm, *refs):
        idx_v = refs[0:_NBUF]
        rows_v = refs[_NBUF:2 * _NBUF]
        gsem = refs[2 * _NBUF:3 * _NBUF]
        ssem = refs[3 * _NBUF:4 * _NBUF]

        wid = lax.axis_index("s") * _NC + lax.axis_index("c")
        base = wid * b_per_w

        def out_slice(g):
            off = base + g * _CHUNK
            return out_hbm.at[pl.ds(off, _CHUNK)]

        def fire(g, s, wait_store):
            if wait_store:
                pltpu.make_async_copy(
                    rows_v[s], out_slice(g - _NBUF), ssem[s]).wait()
            off = base + g * _CHUNK
            pltpu.sync_copy(idx_hbm.at[pl.ds(off, _CHUNK)], idx_v[s])
            pltpu.async_copy(table_hbm.at[idx_v[s]], rows_v[s], gsem[s])

        def drain(g, s):
            pltpu.make_async_copy(
                table_hbm.at[idx_v[s]], rows_v[s], gsem[s]).wait()
            pltpu.async_copy(rows_v[s], out_slice(g), ssem[s])

        for s in range(_NBUF):
            fire(s, s, wait_store=False)

        def body(j, carry):
            g0 = j * _NBUF
            for s in range(_NBUF):
                drain(g0 + s, s)
            for s in range(_NBUF):
                fire(g0 + _NBUF + s, s, wait_store=True)
            return carry

        lax.fori_loop(0, n_chunks // _NBUF - 1, body, 0)

        g0 = n_chunks - _NBUF
        for s in range(_NBUF):
            drain(g0 + s, s)
        for s in range(_NBUF):
            pltpu.make_async_copy(
                rows_v[s], out_slice(g0 + s), ssem[s]).wait()

    return gather_k


@functools.lru_cache(maxsize=None)
def _tc_transpose(rows: int, cols: int, bc: int):
    w = cols // 128

    def tkern(x_ref, o_ref):
        x = x_ref[...].reshape(bc, w, 128)
        o_ref[...] = jnp.transpose(x, (1, 2, 0)).reshape(cols, bc)

    return pl.pallas_call(
        tkern,
        grid=(rows // bc,),
        in_specs=[pl.BlockSpec((bc * w, 128), lambda i: (i, 0))],
        out_specs=pl.BlockSpec((cols, bc), lambda i: (0, i)),
        out_shape=jax.ShapeDtypeStruct((cols, rows), jnp.float32),
    )


def kernel(a, table):
    batch, hist = a.shape
    B = batch * hist
    idx = a.reshape(B).astype(jnp.int32)
    out = _build(B)(idx, table)
    x = out.reshape(batch * hist * _EMBED // 128, 128)
    t = _tc_transpose(batch, hist * _EMBED, 128)(x)
    return jnp.transpose(t.reshape(hist, _EMBED, batch), (2, 0, 1))
